# baseline (speedup 1.0000x reference)
import jax
import jax.numpy as jnp
from jax import lax
from jax.experimental import pallas as pl
from jax.experimental.pallas import tpu as pltpu
from jax.experimental.pallas import tpu_sc as plsc

BATCH = 4096
HIST = 200
EMB_DIM = 64
NUM_Y = 16
NC = 2
NS = 16
NW = NC * NS
BAGS_PER_W = BATCH // NW
SPLIT = 104
NPART = 4
PART_D = EMB_DIM // NPART


def _sc_bag_sum(text_hbm, emb_hbm, out_hbm,
                idx_v, buf0, buf1, out_v, sem0, sem1, lin_sem):
    wid = lax.axis_index("s") * NC + lax.axis_index("c")
    base = wid * BAGS_PER_W

    cp = pltpu.make_async_copy(text_hbm.at[pl.ds(base, BAGS_PER_W)], idx_v,
                               lin_sem)
    cp.start()
    cp.wait()

    bufs = (buf0, buf1)
    sems = (sem0, sem1)

    def issue(bag, buf, sem):
        pltpu.make_async_copy(
            emb_hbm.at[idx_v.at[bag, pl.ds(0, SPLIT)]],
            buf.at[pl.ds(0, SPLIT)], sem).start()
        pltpu.make_async_copy(
            emb_hbm.at[idx_v.at[bag, pl.ds(SPLIT, HIST - SPLIT)]],
            buf.at[pl.ds(SPLIT, HIST - SPLIT)], sem).start()

    def wait(buf, sem):
        pltpu.make_async_copy(emb_hbm.at[pl.ds(0, HIST)], buf, sem).wait()

    def compute(bag, buf):
        def acc_body(r, a):
            return a + buf[r, pl.ds(0, PART_D)]
        z = jnp.zeros((PART_D,), jnp.float32)
        out_v[bag, pl.ds(0, PART_D)] = lax.fori_loop(0, HIST, acc_body, z)

    issue(0, bufs[0], sems[0])

    def outer(i, carry):
        g = 2 * i
        issue(g + 1, bufs[1], sems[1])
        wait(bufs[0], sems[0])
        compute(g, bufs[0])

        @pl.when(g + 2 < BAGS_PER_W)
        def _():
            issue(g + 2, bufs[0], sems[0])
        wait(bufs[1], sems[1])
        compute(g + 1, bufs[1])
        return carry

    lax.fori_loop(0, BAGS_PER_W // 2, outer, 0)

    cp = pltpu.make_async_copy(out_v, out_hbm.at[pl.ds(base, BAGS_PER_W)],
                               lin_sem)
    cp.start()
    cp.wait()


def _part_sums(text, emb_part):
    mesh = plsc.VectorSubcoreMesh(core_axis_name="c", subcore_axis_name="s")
    return pl.kernel(
        _sc_bag_sum,
        out_type=jax.ShapeDtypeStruct((BATCH, PART_D), jnp.float32),
        mesh=mesh,
        scratch_types=[
            pltpu.VMEM((BAGS_PER_W, HIST), jnp.int32),
            pltpu.VMEM((HIST, PART_D), jnp.float32),
            pltpu.VMEM((HIST, PART_D), jnp.float32),
            pltpu.VMEM((BAGS_PER_W, PART_D), jnp.float32),
            pltpu.SemaphoreType.DMA,
            pltpu.SemaphoreType.DMA,
            pltpu.SemaphoreType.DMA,
        ],
        compiler_params=pltpu.CompilerParams(use_tc_tiling_on_sc=False),
    )(text, emb_part)


def _tc_linear(s0, s1, s2, s3, w_ref, b_ref, out_ref):
    acc = b_ref[...] + jnp.dot(s0[...], w_ref[pl.ds(0, PART_D), :],
                               preferred_element_type=jnp.float32)
    acc += jnp.dot(s1[...], w_ref[pl.ds(PART_D, PART_D), :],
                   preferred_element_type=jnp.float32)
    acc += jnp.dot(s2[...], w_ref[pl.ds(2 * PART_D, PART_D), :],
                   preferred_element_type=jnp.float32)
    acc += jnp.dot(s3[...], w_ref[pl.ds(3 * PART_D, PART_D), :],
                   preferred_element_type=jnp.float32)
    out_ref[...] = acc


@jax.jit
def _run(text, emb_weight, w_scaled, lin_b):
    sums = [_part_sums(text, emb_weight[:, k * PART_D:(k + 1) * PART_D])
            for k in range(NPART)]
    return pl.pallas_call(
        _tc_linear,
        out_shape=jax.ShapeDtypeStruct((BATCH, NUM_Y), jnp.float32),
    )(*sums, w_scaled, lin_b.reshape(1, NUM_Y))


def kernel(text, emb_weight, lin_w, lin_b):
    w_scaled = lin_w.T.reshape(EMB_DIM, NUM_Y) * jnp.float32(1.0 / HIST)
    return _run(text.astype(jnp.int32), emb_weight, w_scaled, lin_b)


# speedup vs baseline: 2.6890x; 2.6890x over previous
import jax
import jax.numpy as jnp
import numpy as np
from jax import lax
from jax.experimental import pallas as pl
from jax.experimental.pallas import tpu as pltpu
from jax.experimental.pallas import tpu_sc as plsc

BATCH = 4096
HIST = 200
EMB_DIM = 64
NUM_Y = 16
NC = 2
NS = 16
NW = NC * NS
BAGS_PER_W = BATCH // NW
SPLIT = 104

PERM = np.concatenate([np.arange(0, 32, 2), np.arange(1, 32, 2),
                       np.arange(32, 64, 2), np.arange(33, 64, 2)])


def _sc_bag_sum(text_hbm, emb_hbm, out_hbm,
                idx_v, buf0, buf1, out_v, sem0, sem1, lin_sem):
    wid = lax.axis_index("s") * NC + lax.axis_index("c")
    base = wid * BAGS_PER_W

    cp = pltpu.make_async_copy(text_hbm.at[pl.ds(base, BAGS_PER_W)], idx_v,
                               lin_sem)
    cp.start()
    cp.wait()

    bufs = (buf0, buf1)
    sems = (sem0, sem1)

    def issue(bag, buf, sem):
        pltpu.make_async_copy(
            emb_hbm.at[idx_v.at[bag, pl.ds(0, SPLIT)]],
            buf.at[pl.ds(0, SPLIT)], sem).start()
        pltpu.make_async_copy(
            emb_hbm.at[idx_v.at[bag, pl.ds(SPLIT, HIST - SPLIT)]],
            buf.at[pl.ds(SPLIT, HIST - SPLIT)], sem).start()

    def wait(buf, sem):
        pltpu.make_async_copy(emb_hbm.at[pl.ds(0, HIST)], buf, sem).wait()

    def compute(bag, buf):
        def acc_body(r, acc):
            a0, a1, a2, a3 = acc
            lo = buf[r, pl.ds(0, 32)]
            hi = buf[r, pl.ds(32, 32)]
            e0, e1 = plsc.unpack(lo, format=plsc.PackFormat.INTERLEAVED)
            e2, e3 = plsc.unpack(hi, format=plsc.PackFormat.INTERLEAVED)
            return (a0 + e0, a1 + e1, a2 + e2, a3 + e3)
        z = jnp.zeros((16,), jnp.float32)
        a0, a1, a2, a3 = lax.fori_loop(0, HIST, acc_body, (z, z, z, z))
        out_v[bag, pl.ds(0, 16)] = a0
        out_v[bag, pl.ds(16, 16)] = a1
        out_v[bag, pl.ds(32, 16)] = a2
        out_v[bag, pl.ds(48, 16)] = a3

    issue(0, bufs[0], sems[0])

    def outer(i, carry):
        g = 2 * i
        issue(g + 1, bufs[1], sems[1])
        wait(bufs[0], sems[0])
        compute(g, bufs[0])

        @pl.when(g + 2 < BAGS_PER_W)
        def _():
            issue(g + 2, bufs[0], sems[0])
        wait(bufs[1], sems[1])
        compute(g + 1, bufs[1])
        return carry

    lax.fori_loop(0, BAGS_PER_W // 2, outer, 0)

    cp = pltpu.make_async_copy(out_v, out_hbm.at[pl.ds(base, BAGS_PER_W)],
                               lin_sem)
    cp.start()
    cp.wait()


def _tc_linear(sums_ref, w_ref, b_ref, out_ref):
    out_ref[...] = (
        jnp.dot(sums_ref[...], w_ref[...], preferred_element_type=jnp.float32)
        + b_ref[...]
    )


@jax.jit
def _run(text, emb_bf, w_scaled, lin_b):
    mesh = plsc.VectorSubcoreMesh(core_axis_name="c", subcore_axis_name="s")
    bag_sums = pl.kernel(
        _sc_bag_sum,
        out_type=jax.ShapeDtypeStruct((BATCH, EMB_DIM), jnp.float32),
        mesh=mesh,
        scratch_types=[
            pltpu.VMEM((BAGS_PER_W, HIST), jnp.int32),
            pltpu.VMEM((HIST, EMB_DIM), jnp.bfloat16),
            pltpu.VMEM((HIST, EMB_DIM), jnp.bfloat16),
            pltpu.VMEM((BAGS_PER_W, EMB_DIM), jnp.float32),
            pltpu.SemaphoreType.DMA,
            pltpu.SemaphoreType.DMA,
            pltpu.SemaphoreType.DMA,
        ],
        compiler_params=pltpu.CompilerParams(use_tc_tiling_on_sc=False,
                                             needs_layout_passes=False),
    )(text, emb_bf)
    return pl.pallas_call(
        _tc_linear,
        out_shape=jax.ShapeDtypeStruct((BATCH, NUM_Y), jnp.float32),
    )(bag_sums, w_scaled, lin_b.reshape(1, NUM_Y))


def kernel(text, emb_weight, lin_w, lin_b):
    w_perm = (lin_w.T.reshape(EMB_DIM, NUM_Y) * jnp.float32(1.0 / HIST))[PERM]
    return _run(text.astype(jnp.int32), emb_weight.astype(jnp.bfloat16),
                w_perm, lin_b)


# speedup vs baseline: 5.2023x; 1.9346x over previous
import jax
import jax.numpy as jnp
from jax import lax
from jax.experimental import pallas as pl
from jax.experimental import layout as jex_layout
from jax.experimental.pallas import tpu as pltpu
from jax.experimental.pallas import tpu_sc as plsc

BATCH = 4096
HIST = 200
EMB_DIM = 64
NUM_Y = 16
NC = 2
NS = 16
NW = NC * NS
BAGS_PER_W = BATCH // NW
SPLIT = 104


def _sc_bag_sum(text_hbm, emb_hbm, out_hbm,
                idx_v, buf0, buf1, out_v, sem0, sem1, lin_sem):
    wid = lax.axis_index("s") * NC + lax.axis_index("c")
    base = wid * BAGS_PER_W

    cp = pltpu.make_async_copy(text_hbm.at[pl.ds(base, BAGS_PER_W)], idx_v,
                               lin_sem)
    cp.start()
    cp.wait()

    bufs = (buf0, buf1)
    sems = (sem0, sem1)

    def issue(bag, buf, sem):
        pltpu.make_async_copy(
            emb_hbm.at[idx_v.at[bag, pl.ds(0, SPLIT)]],
            buf.at[pl.ds(0, SPLIT)], sem).start()
        pltpu.make_async_copy(
            emb_hbm.at[idx_v.at[bag, pl.ds(SPLIT, HIST - SPLIT)]],
            buf.at[pl.ds(SPLIT, HIST - SPLIT)], sem).start()

    def wait(buf, sem):
        pltpu.make_async_copy(emb_hbm.at[pl.ds(0, HIST)], buf, sem).wait()

    def compute(bag, buf):
        def acc_body(r, acc):
            a0, a1, a2, a3 = acc
            return (a0 + buf[r, pl.ds(0, 16)],
                    a1 + buf[r, pl.ds(16, 16)],
                    a2 + buf[r, pl.ds(32, 16)],
                    a3 + buf[r, pl.ds(48, 16)])
        z = jnp.zeros((16,), jnp.float32)
        a0, a1, a2, a3 = lax.fori_loop(0, HIST, acc_body, (z, z, z, z))
        out_v[bag, pl.ds(0, 16)] = a0
        out_v[bag, pl.ds(16, 16)] = a1
        out_v[bag, pl.ds(32, 16)] = a2
        out_v[bag, pl.ds(48, 16)] = a3

    issue(0, bufs[0], sems[0])

    def outer(i, carry):
        g = 2 * i
        issue(g + 1, bufs[1], sems[1])
        wait(bufs[0], sems[0])
        compute(g, bufs[0])

        @pl.when(g + 2 < BAGS_PER_W)
        def _():
            issue(g + 2, bufs[0], sems[0])
        wait(bufs[1], sems[1])
        compute(g + 1, bufs[1])
        return carry

    lax.fori_loop(0, BAGS_PER_W // 2, outer, 0)

    cp = pltpu.make_async_copy(out_v, out_hbm.at[pl.ds(base, BAGS_PER_W)],
                               lin_sem)
    cp.start()
    cp.wait()


def _tc_linear(sums_ref, w_ref, b_ref, out_ref):
    out_ref[...] = (
        jnp.dot(sums_ref[...], w_ref[...], preferred_element_type=jnp.float32)
        + b_ref[...]
    )


@jax.jit
def _run(text, emb_weight, w_scaled, lin_b):
    emb_row = jex_layout.with_layout_constraint(
        emb_weight, jex_layout.Layout((0, 1), tiling=((8,),)))
    mesh = plsc.VectorSubcoreMesh(core_axis_name="c", subcore_axis_name="s")
    bag_sums = pl.kernel(
        _sc_bag_sum,
        out_type=jax.ShapeDtypeStruct((BATCH, EMB_DIM), jnp.float32),
        mesh=mesh,
        scratch_types=[
            pltpu.VMEM((BAGS_PER_W, HIST), jnp.int32),
            pltpu.VMEM((HIST, EMB_DIM), jnp.float32),
            pltpu.VMEM((HIST, EMB_DIM), jnp.float32),
            pltpu.VMEM((BAGS_PER_W, EMB_DIM), jnp.float32),
            pltpu.SemaphoreType.DMA,
            pltpu.SemaphoreType.DMA,
            pltpu.SemaphoreType.DMA,
        ],
        compiler_params=pltpu.CompilerParams(use_tc_tiling_on_sc=False),
    )(text, emb_row)
    return pl.pallas_call(
        _tc_linear,
        out_shape=jax.ShapeDtypeStruct((BATCH, NUM_Y), jnp.float32),
    )(bag_sums, w_scaled, lin_b.reshape(1, NUM_Y))


def kernel(text, emb_weight, lin_w, lin_b):
    w_scaled = lin_w.T.reshape(EMB_DIM, NUM_Y) * jnp.float32(1.0 / HIST)
    return _run(text.astype(jnp.int32), emb_weight, w_scaled, lin_b)
